# baseline (device time: 47793 ns/iter reference)
import jax
import jax.numpy as jnp
from jax import lax
from jax.experimental import pallas as pl
from jax.experimental.pallas import tpu as pltpu

N_DEV = 32
N_LAYERS = 3
STAGES = ((4, 0), (2, 2), (4, 3))
SLOTS_PER_LAYER = sum(r - 1 for r, _ in STAGES)
N_SLOTS = N_LAYERS * SLOTS_PER_LAYER
N_ROUNDS = N_LAYERS * len(STAGES)

BF16 = jnp.bfloat16
F32 = jnp.float32


def kernel(x, Win0, Wout0, Win1, Wout1, Win2, Wout2):
    b, d_shard = x.shape
    h_dim = Win0.shape[1]

    def body(x_ref, win0_ref, wout0_ref, win1_ref, wout1_ref, win2_ref,
             wout2_ref, out_ref, send_ref, recv_ref, send_sems, recv_sems):
        my_pos = lax.axis_index("i")

        state = {"slot": 0, "round": 0, "pending": {}}

        def exchange_round(acc, radix, shift, overlap_fn=None):
            r = state["round"]
            sb = send_ref.at[r % 2]
            for d in state["pending"].pop(r - 2, ()):
                d.wait_send()
            sb[...] = acc.astype(BF16)
            rdmas = []
            for j in range(1, radix):
                s = state["slot"]
                rdma = pltpu.make_async_remote_copy(
                    src_ref=sb,
                    dst_ref=recv_ref.at[s],
                    send_sem=send_sems.at[s],
                    recv_sem=recv_sems.at[s],
                    device_id=(my_pos ^ (j << shift),),
                    device_id_type=pl.DeviceIdType.MESH,
                )
                rdma.start()
                rdmas.append((s, rdma))
                state["slot"] += 1
            state["pending"][r] = [d for _, d in rdmas]
            state["round"] += 1
            hidden = overlap_fn() if overlap_fn is not None else None
            for s, rdma in rdmas:
                rdma.wait_recv()
                acc = acc + recv_ref[s].astype(F32)
            return acc, hidden

        xb = x_ref[...].astype(BF16)
        acc = jnp.dot(xb, win0_ref[...].astype(BF16),
                      preferred_element_type=F32)

        barrier_sem = pltpu.get_barrier_semaphore()
        n_partners = 0
        for radix, shift in STAGES:
            for j in range(1, radix):
                pl.semaphore_signal(
                    barrier_sem, inc=1,
                    device_id=(my_pos ^ (j << shift),),
                    device_id_type=pl.DeviceIdType.MESH,
                )
                n_partners += 1
        pl.semaphore_wait(barrier_sem, n_partners)

        def make_w01():
            return jnp.dot(wout0_ref[...].astype(BF16),
                           win1_ref[...].astype(BF16),
                           preferred_element_type=F32).astype(BF16)

        def make_w12():
            return jnp.dot(wout1_ref[...].astype(BF16),
                           win2_ref[...].astype(BF16),
                           preferred_element_type=F32).astype(BF16)

        acc, w01 = exchange_round(acc, *STAGES[0], overlap_fn=make_w01)
        acc, w12 = exchange_round(acc, *STAGES[1], overlap_fn=make_w12)
        acc, wout2b = exchange_round(
            acc, *STAGES[2],
            overlap_fn=lambda: wout2_ref[...].astype(BF16))

        h = jnp.maximum(acc, 0.0).astype(BF16)
        acc = jnp.dot(h, w01, preferred_element_type=F32)
        for radix, shift in STAGES:
            acc, _ = exchange_round(acc, radix, shift)

        h = jnp.maximum(acc, 0.0).astype(BF16)
        acc = jnp.dot(h, w12, preferred_element_type=F32)
        for radix, shift in STAGES:
            acc, _ = exchange_round(acc, radix, shift)

        h = jnp.maximum(acc, 0.0).astype(BF16)
        out_ref[...] = jnp.dot(h, wout2b, preferred_element_type=F32)

        for r in sorted(state["pending"]):
            for d in state["pending"][r]:
                d.wait_send()

    return pl.pallas_call(
        body,
        out_shape=jax.ShapeDtypeStruct((b, d_shard), jnp.float32),
        in_specs=[pl.BlockSpec(memory_space=pltpu.VMEM)] * 7,
        out_specs=pl.BlockSpec(memory_space=pltpu.VMEM),
        scratch_shapes=[
            pltpu.VMEM((2, b, h_dim), BF16),
            pltpu.VMEM((N_SLOTS, b, h_dim), BF16),
            pltpu.SemaphoreType.DMA((N_SLOTS,)),
            pltpu.SemaphoreType.DMA((N_SLOTS,)),
        ],
        compiler_params=pltpu.CompilerParams(collective_id=0),
    )(x, Win0, Wout0, Win1, Wout1, Win2, Wout2)
